# Initial kernel scaffold; baseline (speedup 1.0000x reference)
import jax
import jax.numpy as jnp
from jax.experimental import pallas as pl


def kernel(boxes, classification):
    raise NotImplementedError("write your pallas kernel here")


# speedup vs baseline: 27.5839x; 27.5839x over previous
import jax
import jax.numpy as jnp
import numpy as np
from jax import lax
from jax.experimental import pallas as pl
from jax.experimental.pallas import tpu as pltpu
from jax.experimental.pallas import tpu_sc as plsc

NEG = np.float32(-1e9)
SCORE_THR = np.float32(0.05)
HALF = np.float32(0.5)
VALID_CUT = np.float32(-5e8)
BIG = np.int32(1 << 30)

N = 5000
NP = 5024
L = 16
NCH = NP // L
CMAXP = 320
NCV = CMAXP // L
C = 20
B = 2
MAXD = 300
MD_P = 304
KV = MD_P // L
NC_ = 2
NS_ = 16


def _lanes():
    return lax.broadcasted_iota(jnp.int32, (L,), 0)


def _splat_i(x):
    return jnp.zeros((L,), jnp.int32) + x


def _splat_f(x):
    return jnp.zeros((L,), jnp.float32) + x


def _scalar(x):
    return x if x.ndim == 0 else jnp.max(x)


def _nms_body(scores_hbm, boxes_hbm, sel_hbm, ssc_hbm,
              sraw, s, bx, area, cmax, kept, selv, sscv):
    cid = lax.axis_index("c")
    sid = lax.axis_index("s")
    wid = sid * NC_ + cid
    lanes = _lanes()
    lane0 = lanes == 0

    def run_task(task):
        img = task // C
        klass = lax.rem(task, C)
        pltpu.sync_copy(scores_hbm.at[img, klass], sraw)
        pltpu.sync_copy(boxes_hbm.at[img], bx)

        for i in range(NCV):
            cmax[pl.ds(i * L, L)] = jnp.full((L,), NEG, jnp.float32)

        def init_out(i, carry):
            selv[pl.ds(i * L, L)] = jnp.full((L,), -1, jnp.int32)
            sscv[pl.ds(i * L, L)] = jnp.full((L,), -1.0, jnp.float32)
            kept[0, pl.ds(i * L, L)] = jnp.full((L,), 3.0, jnp.float32)
            kept[1, pl.ds(i * L, L)] = jnp.full((L,), 3.0, jnp.float32)
            kept[2, pl.ds(i * L, L)] = jnp.full((L,), 0.0, jnp.float32)
            kept[3, pl.ds(i * L, L)] = jnp.full((L,), 0.0, jnp.float32)
            kept[4, pl.ds(i * L, L)] = jnp.full((L,), 0.0, jnp.float32)
            return carry

        lax.fori_loop(0, KV, init_out, 0)

        def init_chunk(i, carry):
            v = sraw[pl.ds(i * L, L)]
            m = jnp.where(v > SCORE_THR, v, NEG)
            s[pl.ds(i * L, L)] = m
            y1 = bx[0, pl.ds(i * L, L)]
            x1 = bx[1, pl.ds(i * L, L)]
            y2 = bx[2, pl.ds(i * L, L)]
            x2 = bx[3, pl.ds(i * L, L)]
            a = jnp.maximum(y2 - y1, jnp.float32(0.0)) * \
                jnp.maximum(x2 - x1, jnp.float32(0.0))
            area[pl.ds(i * L, L)] = a
            plsc.store_scatter(cmax, [_splat_i(i)], _splat_f(jnp.max(m)),
                               mask=lane0)
            return carry

        lax.fori_loop(0, NCH, init_chunk, 0)

        def step(t, carry):
            K, exh = carry

            def active(_):
                def wcond(cw):
                    return cw[0] == 0

                def wbody(cw):
                    _, K2, exh2 = cw

                    def am(i, mc):
                        m, ci = mc
                        v = cmax[pl.ds(i * L, L)]
                        upd = v > m
                        return (jnp.where(upd, v, m),
                                jnp.where(upd, _splat_i(i), ci))

                    m, ci = lax.fori_loop(
                        0, NCV, am,
                        (jnp.full((L,), NEG, jnp.float32),
                         jnp.zeros((L,), jnp.int32)))
                    M = jnp.max(m)
                    gc = jnp.where(m == M, ci * L + lanes, BIG)
                    cstar = jnp.min(gc)
                    valid = M > VALID_CUT

                    def found(_):
                        v = s[pl.ds(cstar * L, L)]
                        lane = _scalar(plsc.all_reduce_ffs(v == M))
                        idx = cstar * L + lane
                        idxv = _splat_i(idx)
                        cy1 = plsc.load_gather(bx, [_splat_i(0), idxv])
                        cx1 = plsc.load_gather(bx, [_splat_i(1), idxv])
                        cy2 = plsc.load_gather(bx, [_splat_i(2), idxv])
                        cx2 = plsc.load_gather(bx, [_splat_i(3), idxv])
                        ca = plsc.load_gather(area, [idxv])
                        plsc.store_scatter(
                            s, [idxv], jnp.full((L,), NEG, jnp.float32),
                            mask=lane0)
                        v2 = s[pl.ds(cstar * L, L)]
                        plsc.store_scatter(cmax, [_splat_i(cstar)],
                                           _splat_f(jnp.max(v2)), mask=lane0)

                        def vb(j, acc):
                            ky1 = kept[0, pl.ds(j * L, L)]
                            kx1 = kept[1, pl.ds(j * L, L)]
                            ky2 = kept[2, pl.ds(j * L, L)]
                            kx2 = kept[3, pl.ds(j * L, L)]
                            ka = kept[4, pl.ds(j * L, L)]
                            yy1 = jnp.maximum(cy1, ky1)
                            xx1 = jnp.maximum(cx1, kx1)
                            yy2 = jnp.minimum(cy2, ky2)
                            xx2 = jnp.minimum(cx2, kx2)
                            inter = jnp.maximum(yy2 - yy1, jnp.float32(0.0)) * \
                                jnp.maximum(xx2 - xx1, jnp.float32(0.0))
                            rhs = HALF * ((ca + ka) - inter)
                            return jnp.logical_or(acc, inter > rhs)

                        acc = lax.fori_loop(0, KV, vb,
                                            jnp.zeros((L,), jnp.bool_))
                        sup = jnp.any(acc)

                        def keep(_):
                            plsc.store_scatter(kept, [_splat_i(0), _splat_i(K2)],
                                               cy1, mask=lane0)
                            plsc.store_scatter(kept, [_splat_i(1), _splat_i(K2)],
                                               cx1, mask=lane0)
                            plsc.store_scatter(kept, [_splat_i(2), _splat_i(K2)],
                                               cy2, mask=lane0)
                            plsc.store_scatter(kept, [_splat_i(3), _splat_i(K2)],
                                               cx2, mask=lane0)
                            plsc.store_scatter(kept, [_splat_i(4), _splat_i(K2)],
                                               ca, mask=lane0)
                            plsc.store_scatter(selv, [_splat_i(t)], idxv,
                                               mask=lane0)
                            plsc.store_scatter(sscv, [_splat_i(t)], _splat_f(M),
                                               mask=lane0)
                            return (jnp.int32(1), K2 + jnp.int32(1), exh2)

                        def rej(_):
                            return (jnp.int32(0), K2, exh2)

                        return lax.cond(sup, rej, keep, 0)

                    def nomore(_):
                        return (jnp.int32(1), K2, jnp.int32(1))

                    return lax.cond(valid, found, nomore, 0)

                _, K3, exh3 = lax.while_loop(
                    wcond, wbody, (jnp.int32(0), K, exh))
                return (K3, exh3)

            def skip(_):
                return (K, exh)

            return lax.cond(exh != 0, skip, active, 0)

        lax.fori_loop(0, MAXD, step, (jnp.int32(0), jnp.int32(0)))
        pltpu.sync_copy(selv, sel_hbm.at[img, klass])
        pltpu.sync_copy(sscv, ssc_hbm.at[img, klass])

    run_task(wid)

    @pl.when(wid + 32 < B * C)
    def _second():
        run_task(wid + 32)


def _merge_body(selsc_hbm, sel_hbm, boxes_hbm, ob_hbm, os_hbm, ol_hbm,
                sscv, selv, bxv, heads, outb, outs, outl):
    cid = lax.axis_index("c")
    sid = lax.axis_index("s")
    wid = sid * NC_ + cid
    lanes = _lanes()
    lane0 = lanes == 0

    @pl.when(wid < B)
    def _():
        img = wid
        pltpu.sync_copy(selsc_hbm.at[img], sscv)
        pltpu.sync_copy(sel_hbm.at[img], selv)
        pltpu.sync_copy(boxes_hbm.at[img], bxv)
        heads[pl.ds(0, L)] = jnp.zeros((L,), jnp.int32)
        heads[pl.ds(L, L)] = jnp.zeros((L,), jnp.int32)

        def io(i, carry):
            outs[pl.ds(i * L, L)] = jnp.full((L,), -1.0, jnp.float32)
            outl[pl.ds(i * L, L)] = jnp.full((L,), -1, jnp.int32)
            return carry

        lax.fori_loop(0, KV, io, 0)

        def iob(i, carry):
            outb[pl.ds(i * L, L)] = jnp.full((L,), -1.0, jnp.float32)
            return carry

        lax.fori_loop(0, MD_P * 4 // L, iob, 0)

        def step(t, carry):
            hl = plsc.load_gather(heads, [lanes])
            g_lo = plsc.load_gather(sscv, [lanes, hl])
            rhi = jnp.minimum(lanes + L, jnp.int32(C - 1))
            hh = plsc.load_gather(heads, [lanes + L])
            g_hi_raw = plsc.load_gather(sscv, [rhi, hh])
            g_hi = jnp.where(lanes < C - L, g_hi_raw, jnp.float32(-2.0))
            comb = jnp.maximum(g_lo, g_hi)
            M = jnp.max(comb)
            in_lo = jnp.any(g_lo == M)
            lane_lo = _scalar(plsc.all_reduce_ffs(g_lo == M))
            lane_hi = _scalar(plsc.all_reduce_ffs(g_hi == M))
            cls = lax.select(in_lo, lane_lo, lane_hi + jnp.int32(L))
            clsv = _splat_i(cls)
            hv = plsc.load_gather(heads, [clsv])
            plsc.store_scatter(heads, [clsv], hv + 1, mask=lane0)
            bi = plsc.load_gather(selv, [clsv, hv])
            vv = bi >= 0
            safe = jnp.maximum(bi, 0)
            coord = lax.rem(lanes, jnp.int32(4))
            bvals = plsc.load_gather(bxv, [safe, coord])
            obv = jnp.where(vv, bvals, jnp.float32(-1.0))
            plsc.store_scatter(outb, [_splat_i(t * 4) + coord], obv,
                               mask=lanes < 4)
            plsc.store_scatter(outs, [_splat_i(t)],
                               jnp.where(vv, _splat_f(M), jnp.float32(-1.0)),
                               mask=lane0)
            plsc.store_scatter(outl, [_splat_i(t)],
                               jnp.where(vv, clsv, jnp.int32(-1)), mask=lane0)
            return carry

        lax.fori_loop(0, MAXD, step, 0)
        pltpu.sync_copy(outb, ob_hbm.at[img])
        pltpu.sync_copy(outs, os_hbm.at[img])
        pltpu.sync_copy(outl, ol_hbm.at[img])


_mesh = plsc.VectorSubcoreMesh(core_axis_name="c", subcore_axis_name="s")

_nms_call = pl.kernel(
    _nms_body,
    out_type=(jax.ShapeDtypeStruct((B, C, MD_P), jnp.int32),
              jax.ShapeDtypeStruct((B, C, MD_P), jnp.float32)),
    mesh=_mesh,
    compiler_params=pltpu.CompilerParams(needs_layout_passes=False, use_tc_tiling_on_sc=False),
    scratch_types=[
        pltpu.VMEM((NP,), jnp.float32),
        pltpu.VMEM((NP,), jnp.float32),
        pltpu.VMEM((4, NP), jnp.float32),
        pltpu.VMEM((NP,), jnp.float32),
        pltpu.VMEM((CMAXP,), jnp.float32),
        pltpu.VMEM((5, MD_P), jnp.float32),
        pltpu.VMEM((MD_P,), jnp.int32),
        pltpu.VMEM((MD_P,), jnp.float32),
    ],
)

_merge_call = pl.kernel(
    _merge_body,
    out_type=(jax.ShapeDtypeStruct((B, MD_P * 4), jnp.float32),
              jax.ShapeDtypeStruct((B, MD_P), jnp.float32),
              jax.ShapeDtypeStruct((B, MD_P), jnp.int32)),
    mesh=_mesh,
    compiler_params=pltpu.CompilerParams(needs_layout_passes=False, use_tc_tiling_on_sc=False),
    scratch_types=[
        pltpu.VMEM((C, MD_P), jnp.float32),
        pltpu.VMEM((C, MD_P), jnp.int32),
        pltpu.VMEM((N, 4), jnp.float32),
        pltpu.VMEM((2 * L,), jnp.int32),
        pltpu.VMEM((MD_P * 4,), jnp.float32),
        pltpu.VMEM((MD_P,), jnp.float32),
        pltpu.VMEM((MD_P,), jnp.int32),
    ],
)


def kernel(boxes, classification):
    b = boxes.astype(jnp.float32)
    c = classification.astype(jnp.float32)
    scores_t = jnp.pad(jnp.transpose(c, (0, 2, 1)),
                       ((0, 0), (0, 0), (0, NP - N)))
    boxes_t = jnp.pad(jnp.transpose(b, (0, 2, 1)),
                      ((0, 0), (0, 0), (0, NP - N)))
    sel, ssc = _nms_call(scores_t, boxes_t)
    ob, osc, ol = _merge_call(ssc, sel, b)
    ob = ob.reshape(B, MD_P, 4)[:, :MAXD, :]
    return ob, osc[:, :MAXD], ol[:, :MAXD]


# speedup vs baseline: 42.7585x; 1.5501x over previous
import jax
import jax.numpy as jnp
import numpy as np
from jax import lax
from jax.experimental import pallas as pl
from jax.experimental.pallas import tpu as pltpu
from jax.experimental.pallas import tpu_sc as plsc

NEG = np.float32(-1e9)
SCORE_THR = np.float32(0.05)
HALF = np.float32(0.5)
VALID_CUT = np.float32(-5e8)
BIG = np.int32(1 << 30)

N = 5000
NP = 5120
L = 16
NCH = NP // L
CMAXP = 320
NCV = CMAXP // L
C = 20
B = 2
MAXD = 300
MD_P = 304
KV = MD_P // L
NC_ = 2
NS_ = 16


def _lanes():
    return lax.broadcasted_iota(jnp.int32, (L,), 0)


def _splat_i(x):
    return jnp.zeros((L,), jnp.int32) + x


def _splat_f(x):
    return jnp.zeros((L,), jnp.float32) + x


def _scalar(x):
    return x if x.ndim == 0 else jnp.max(x)


def _nms_body(scores_hbm, boxes_hbm, sel_hbm, ssc_hbm,
              sraw, s, bx, area, cmax, kept, selv, sscv):
    cid = lax.axis_index("c")
    sid = lax.axis_index("s")
    wid = sid * NC_ + cid
    lanes = _lanes()
    lane0 = lanes == 0

    def run_task(task):
        img = task // C
        klass = lax.rem(task, C)
        pltpu.sync_copy(scores_hbm.at[img, klass], sraw)
        pltpu.sync_copy(boxes_hbm.at[img], bx)

        def init_out(i, carry):
            selv[pl.ds(i * L, L)] = jnp.full((L,), -1, jnp.int32)
            sscv[pl.ds(i * L, L)] = jnp.full((L,), -1.0, jnp.float32)
            kept[0, pl.ds(i * L, L)] = jnp.full((L,), 3.0, jnp.float32)
            kept[1, pl.ds(i * L, L)] = jnp.full((L,), 3.0, jnp.float32)
            kept[2, pl.ds(i * L, L)] = jnp.full((L,), 0.0, jnp.float32)
            kept[3, pl.ds(i * L, L)] = jnp.full((L,), 0.0, jnp.float32)
            kept[4, pl.ds(i * L, L)] = jnp.full((L,), 0.0, jnp.float32)
            return carry

        lax.fori_loop(0, KV, init_out, 0)

        def init_chunk(i, carry):
            for u in range(4):
                c = i * 4 + u
                v = sraw[pl.ds(c * L, L)]
                m = jnp.where(v > SCORE_THR, v, NEG)
                s[pl.ds(c * L, L)] = m
                y1 = bx[0, pl.ds(c * L, L)]
                x1 = bx[1, pl.ds(c * L, L)]
                y2 = bx[2, pl.ds(c * L, L)]
                x2 = bx[3, pl.ds(c * L, L)]
                a = jnp.maximum(y2 - y1, jnp.float32(0.0)) * \
                    jnp.maximum(x2 - x1, jnp.float32(0.0))
                area[pl.ds(c * L, L)] = a
                plsc.store_scatter(cmax, [_splat_i(c)], _splat_f(jnp.max(m)),
                                   mask=lane0)
            return carry

        lax.fori_loop(0, NCH // 4, init_chunk, 0)

        def step(t, carry):
            K, exh = carry

            def active(_):
                def wcond(cw):
                    return cw[0] == 0

                def wbody(cw):
                    _, K2, exh2 = cw

                    m = cmax[pl.ds(0, L)]
                    ci = jnp.zeros((L,), jnp.int32)
                    for i in range(1, NCV):
                        v = cmax[pl.ds(i * L, L)]
                        upd = v > m
                        m = jnp.where(upd, v, m)
                        ci = jnp.where(upd, _splat_i(i), ci)
                    M = jnp.max(m)
                    gc = jnp.where(m == M, ci * L + lanes, BIG)
                    cstar = jnp.min(gc)
                    valid = M > VALID_CUT

                    def found(_):
                        v = s[pl.ds(cstar * L, L)]
                        lane = _scalar(plsc.all_reduce_ffs(v == M))
                        idx = cstar * L + lane
                        idxv = _splat_i(idx)
                        cy1 = plsc.load_gather(bx, [_splat_i(0), idxv])
                        cx1 = plsc.load_gather(bx, [_splat_i(1), idxv])
                        cy2 = plsc.load_gather(bx, [_splat_i(2), idxv])
                        cx2 = plsc.load_gather(bx, [_splat_i(3), idxv])
                        ca = plsc.load_gather(area, [idxv])
                        plsc.store_scatter(
                            s, [idxv], jnp.full((L,), NEG, jnp.float32),
                            mask=lane0)
                        v2 = jnp.where(lanes == lane, NEG, v)
                        plsc.store_scatter(cmax, [_splat_i(cstar)],
                                           _splat_f(jnp.max(v2)), mask=lane0)

                        acc = jnp.zeros((L,), jnp.bool_)
                        for j in range(KV):
                            ky1 = kept[0, pl.ds(j * L, L)]
                            kx1 = kept[1, pl.ds(j * L, L)]
                            ky2 = kept[2, pl.ds(j * L, L)]
                            kx2 = kept[3, pl.ds(j * L, L)]
                            ka = kept[4, pl.ds(j * L, L)]
                            yy1 = jnp.maximum(cy1, ky1)
                            xx1 = jnp.maximum(cx1, kx1)
                            yy2 = jnp.minimum(cy2, ky2)
                            xx2 = jnp.minimum(cx2, kx2)
                            inter = jnp.maximum(yy2 - yy1, jnp.float32(0.0)) * \
                                jnp.maximum(xx2 - xx1, jnp.float32(0.0))
                            rhs = HALF * ((ca + ka) - inter)
                            acc = jnp.logical_or(acc, inter > rhs)
                        sup = jnp.any(acc)

                        def keep(_):
                            plsc.store_scatter(kept, [_splat_i(0), _splat_i(K2)],
                                               cy1, mask=lane0)
                            plsc.store_scatter(kept, [_splat_i(1), _splat_i(K2)],
                                               cx1, mask=lane0)
                            plsc.store_scatter(kept, [_splat_i(2), _splat_i(K2)],
                                               cy2, mask=lane0)
                            plsc.store_scatter(kept, [_splat_i(3), _splat_i(K2)],
                                               cx2, mask=lane0)
                            plsc.store_scatter(kept, [_splat_i(4), _splat_i(K2)],
                                               ca, mask=lane0)
                            plsc.store_scatter(selv, [_splat_i(t)], idxv,
                                               mask=lane0)
                            plsc.store_scatter(sscv, [_splat_i(t)], _splat_f(M),
                                               mask=lane0)
                            return (jnp.int32(1), K2 + jnp.int32(1), exh2)

                        def rej(_):
                            return (jnp.int32(0), K2, exh2)

                        return lax.cond(sup, rej, keep, 0)

                    def nomore(_):
                        return (jnp.int32(1), K2, jnp.int32(1))

                    return lax.cond(valid, found, nomore, 0)

                _, K3, exh3 = lax.while_loop(
                    wcond, wbody, (jnp.int32(0), K, exh))
                return (K3, exh3)

            def skip(_):
                return (K, exh)

            return lax.cond(exh != 0, skip, active, 0)

        lax.fori_loop(0, MAXD, step, (jnp.int32(0), jnp.int32(0)))
        pltpu.sync_copy(selv, sel_hbm.at[img, klass])
        pltpu.sync_copy(sscv, ssc_hbm.at[img, klass])

    run_task(wid)

    @pl.when(wid + 32 < B * C)
    def _second():
        run_task(wid + 32)


def _merge_body(selsc_hbm, sel_hbm, boxes_hbm, ob_hbm, os_hbm, ol_hbm,
                sscv, selv, bxv, heads, outb, outs, outl):
    cid = lax.axis_index("c")
    sid = lax.axis_index("s")
    wid = sid * NC_ + cid
    lanes = _lanes()
    lane0 = lanes == 0

    @pl.when(wid < B)
    def _():
        img = wid
        pltpu.sync_copy(selsc_hbm.at[img], sscv)
        pltpu.sync_copy(sel_hbm.at[img], selv)
        pltpu.sync_copy(boxes_hbm.at[img], bxv)
        heads[pl.ds(0, L)] = jnp.zeros((L,), jnp.int32)
        heads[pl.ds(L, L)] = jnp.zeros((L,), jnp.int32)

        def io(i, carry):
            outs[pl.ds(i * L, L)] = jnp.full((L,), -1.0, jnp.float32)
            outl[pl.ds(i * L, L)] = jnp.full((L,), -1, jnp.int32)
            return carry

        lax.fori_loop(0, KV, io, 0)

        def iob(i, carry):
            outb[pl.ds(i * L, L)] = jnp.full((L,), -1.0, jnp.float32)
            return carry

        lax.fori_loop(0, MD_P * 4 // L, iob, 0)

        def step(t, carry):
            hl = plsc.load_gather(heads, [lanes])
            g_lo = plsc.load_gather(sscv, [lanes, hl])
            rhi = jnp.minimum(lanes + L, jnp.int32(C - 1))
            hh = plsc.load_gather(heads, [lanes + L])
            g_hi_raw = plsc.load_gather(sscv, [rhi, hh])
            g_hi = jnp.where(lanes < C - L, g_hi_raw, jnp.float32(-2.0))
            comb = jnp.maximum(g_lo, g_hi)
            M = jnp.max(comb)
            in_lo = jnp.any(g_lo == M)
            lane_lo = _scalar(plsc.all_reduce_ffs(g_lo == M))
            lane_hi = _scalar(plsc.all_reduce_ffs(g_hi == M))
            cls = lax.select(in_lo, lane_lo, lane_hi + jnp.int32(L))
            clsv = _splat_i(cls)
            hv = plsc.load_gather(heads, [clsv])
            plsc.store_scatter(heads, [clsv], hv + 1, mask=lane0)
            bi = plsc.load_gather(selv, [clsv, hv])
            vv = bi >= 0
            safe = jnp.maximum(bi, 0)
            coord = lax.rem(lanes, jnp.int32(4))
            bvals = plsc.load_gather(bxv, [safe, coord])
            obv = jnp.where(vv, bvals, jnp.float32(-1.0))
            plsc.store_scatter(outb, [_splat_i(t * 4) + coord], obv,
                               mask=lanes < 4)
            plsc.store_scatter(outs, [_splat_i(t)],
                               jnp.where(vv, _splat_f(M), jnp.float32(-1.0)),
                               mask=lane0)
            plsc.store_scatter(outl, [_splat_i(t)],
                               jnp.where(vv, clsv, jnp.int32(-1)), mask=lane0)
            return carry

        lax.fori_loop(0, MAXD, step, 0)
        pltpu.sync_copy(outb, ob_hbm.at[img])
        pltpu.sync_copy(outs, os_hbm.at[img])
        pltpu.sync_copy(outl, ol_hbm.at[img])


_mesh = plsc.VectorSubcoreMesh(core_axis_name="c", subcore_axis_name="s")

_nms_call = pl.kernel(
    _nms_body,
    out_type=(jax.ShapeDtypeStruct((B, C, MD_P), jnp.int32),
              jax.ShapeDtypeStruct((B, C, MD_P), jnp.float32)),
    mesh=_mesh,
    compiler_params=pltpu.CompilerParams(needs_layout_passes=False, use_tc_tiling_on_sc=False),
    scratch_types=[
        pltpu.VMEM((NP,), jnp.float32),
        pltpu.VMEM((NP,), jnp.float32),
        pltpu.VMEM((4, NP), jnp.float32),
        pltpu.VMEM((NP,), jnp.float32),
        pltpu.VMEM((CMAXP,), jnp.float32),
        pltpu.VMEM((5, MD_P), jnp.float32),
        pltpu.VMEM((MD_P,), jnp.int32),
        pltpu.VMEM((MD_P,), jnp.float32),
    ],
)

_merge_call = pl.kernel(
    _merge_body,
    out_type=(jax.ShapeDtypeStruct((B, MD_P * 4), jnp.float32),
              jax.ShapeDtypeStruct((B, MD_P), jnp.float32),
              jax.ShapeDtypeStruct((B, MD_P), jnp.int32)),
    mesh=_mesh,
    compiler_params=pltpu.CompilerParams(needs_layout_passes=False, use_tc_tiling_on_sc=False),
    scratch_types=[
        pltpu.VMEM((C, MD_P), jnp.float32),
        pltpu.VMEM((C, MD_P), jnp.int32),
        pltpu.VMEM((N, 4), jnp.float32),
        pltpu.VMEM((2 * L,), jnp.int32),
        pltpu.VMEM((MD_P * 4,), jnp.float32),
        pltpu.VMEM((MD_P,), jnp.float32),
        pltpu.VMEM((MD_P,), jnp.int32),
    ],
)


def kernel(boxes, classification):
    b = boxes.astype(jnp.float32)
    c = classification.astype(jnp.float32)
    scores_t = jnp.pad(jnp.transpose(c, (0, 2, 1)),
                       ((0, 0), (0, 0), (0, NP - N)))
    boxes_t = jnp.pad(jnp.transpose(b, (0, 2, 1)),
                      ((0, 0), (0, 0), (0, NP - N)))
    sel, ssc = _nms_call(scores_t, boxes_t)
    ob, osc, ol = _merge_call(ssc, sel, b)
    ob = ob.reshape(B, MD_P, 4)[:, :MAXD, :]
    return ob, osc[:, :MAXD], ol[:, :MAXD]


# speedup vs baseline: 56.4635x; 1.3205x over previous
import jax
import jax.numpy as jnp
import numpy as np
from jax import lax
from jax.experimental import pallas as pl
from jax.experimental.pallas import tpu as pltpu
from jax.experimental.pallas import tpu_sc as plsc

NEG = np.float32(-1e9)
SCORE_THR = np.float32(0.05)
HALF = np.float32(0.5)
VALID_CUT = np.float32(-5e8)
BIG = np.int32(1 << 30)

N = 5000
NP = 5120
L = 16
NCH = NP // L
CMAXP = 320
NCV = CMAXP // L
C = 20
B = 2
MAXD = 300
MD_P = 304
KV = MD_P // L
NC_ = 2
NS_ = 16


def _lanes():
    return lax.broadcasted_iota(jnp.int32, (L,), 0)


def _splat_i(x):
    return jnp.zeros((L,), jnp.int32) + x


def _splat_f(x):
    return jnp.zeros((L,), jnp.float32) + x


def _scalar(x):
    return x if x.ndim == 0 else jnp.max(x)


def _as_splat_i(x):
    return _splat_i(x) if x.ndim == 0 else x


def _rot(v, sh):
    lanes = _lanes()
    idx = jnp.bitwise_and(lanes + sh, L - 1)
    return v.at[idx].get(mode="promise_in_bounds")


def _tree_max(v):
    for sh in (8, 4, 2, 1):
        v = jnp.maximum(v, _rot(v, sh))
    return v


def _tree_min(v):
    for sh in (8, 4, 2, 1):
        v = jnp.minimum(v, _rot(v, sh))
    return v


def _nms_body(scores_hbm, boxes_hbm, sel_hbm, ssc_hbm,
              sraw, s, bx, area, cmax, kept, selv, sscv):
    cid = lax.axis_index("c")
    sid = lax.axis_index("s")
    wid = sid * NC_ + cid
    lanes = _lanes()
    lane0 = lanes == 0

    def run_task(task):
        img = task // C
        klass = lax.rem(task, C)
        pltpu.sync_copy(scores_hbm.at[img, klass], sraw)
        pltpu.sync_copy(boxes_hbm.at[img], bx)

        def init_out(i, carry):
            @pl.when(i < KV)
            def _():
                selv[pl.ds(i * L, L)] = jnp.full((L,), -1, jnp.int32)
                sscv[pl.ds(i * L, L)] = jnp.full((L,), -1.0, jnp.float32)
            kept[0, pl.ds(i * L, L)] = jnp.full((L,), 3.0, jnp.float32)
            kept[1, pl.ds(i * L, L)] = jnp.full((L,), 3.0, jnp.float32)
            kept[2, pl.ds(i * L, L)] = jnp.full((L,), 0.0, jnp.float32)
            kept[3, pl.ds(i * L, L)] = jnp.full((L,), 0.0, jnp.float32)
            kept[4, pl.ds(i * L, L)] = jnp.full((L,), 0.0, jnp.float32)
            return carry

        lax.fori_loop(0, KV + 1, init_out, 0)

        def init_chunk(i, carry):
            for u in range(4):
                c = i * 4 + u
                v = sraw[pl.ds(c * L, L)]
                m = jnp.where(v > SCORE_THR, v, NEG)
                s[pl.ds(c * L, L)] = m
                y1 = bx[0, pl.ds(c * L, L)]
                x1 = bx[1, pl.ds(c * L, L)]
                y2 = bx[2, pl.ds(c * L, L)]
                x2 = bx[3, pl.ds(c * L, L)]
                a = jnp.maximum(y2 - y1, jnp.float32(0.0)) * \
                    jnp.maximum(x2 - x1, jnp.float32(0.0))
                area[pl.ds(c * L, L)] = a
                plsc.store_scatter(cmax, [_splat_i(c)], _splat_f(jnp.max(m)),
                                   mask=lane0)
            return carry

        lax.fori_loop(0, NCH // 4, init_chunk, 0)

        def step(t, K):
            def wcond(cw):
                return cw[0] == 0

            def wbody(cw):
                _, K2 = cw

                m = cmax[pl.ds(0, L)]
                ci = jnp.zeros((L,), jnp.int32)
                for i in range(1, NCV):
                    v = cmax[pl.ds(i * L, L)]
                    upd = v > m
                    m = jnp.where(upd, v, m)
                    ci = jnp.where(upd, _splat_i(i), ci)
                Mv = _tree_max(m)
                gc = jnp.where(m == Mv, ci * L + lanes, BIG)
                cstarv = _tree_min(gc)
                validv = Mv > VALID_CUT

                v = plsc.load_gather(s, [cstarv * L + lanes])
                lanev = _as_splat_i(plsc.all_reduce_ffs(v == Mv))
                idxv = cstarv * L + lanev
                cy1 = plsc.load_gather(bx, [jnp.zeros((L,), jnp.int32), idxv])
                cx1 = plsc.load_gather(bx, [_splat_i(1), idxv])
                cy2 = plsc.load_gather(bx, [_splat_i(2), idxv])
                cx2 = plsc.load_gather(bx, [_splat_i(3), idxv])
                ca = plsc.load_gather(area, [idxv])
                rm = lane0 & validv
                plsc.store_scatter(s, [idxv],
                                   jnp.full((L,), NEG, jnp.float32), mask=rm)
                v2 = jnp.where(lanes == lanev, NEG, v)
                plsc.store_scatter(cmax, [cstarv], _tree_max(v2), mask=rm)

                acc = jnp.zeros((L,), jnp.bool_)

                def vb(j, acc):
                    for u in range(2):
                        off = j * 2 * L + u * L
                        ky1 = kept[0, pl.ds(off, L)]
                        kx1 = kept[1, pl.ds(off, L)]
                        ky2 = kept[2, pl.ds(off, L)]
                        kx2 = kept[3, pl.ds(off, L)]
                        ka = kept[4, pl.ds(off, L)]
                        yy1 = jnp.maximum(cy1, ky1)
                        xx1 = jnp.maximum(cx1, kx1)
                        yy2 = jnp.minimum(cy2, ky2)
                        xx2 = jnp.minimum(cx2, kx2)
                        inter = jnp.maximum(yy2 - yy1, jnp.float32(0.0)) * \
                            jnp.maximum(xx2 - xx1, jnp.float32(0.0))
                        rhs = HALF * ((ca + ka) - inter)
                        acc = jnp.logical_or(acc, inter > rhs)
                    return acc

                acc = lax.fori_loop(0, (K2 + 31) // 32, vb, acc)
                supv = _as_splat_i(plsc.all_reduce_population_count(acc)) > 0
                keepv = validv & jnp.logical_not(supv)
                wm = lane0 & keepv
                K2v = _splat_i(K2)
                plsc.store_scatter(kept, [jnp.zeros((L,), jnp.int32), K2v],
                                   cy1, mask=wm)
                plsc.store_scatter(kept, [_splat_i(1), K2v], cx1, mask=wm)
                plsc.store_scatter(kept, [_splat_i(2), K2v], cy2, mask=wm)
                plsc.store_scatter(kept, [_splat_i(3), K2v], cx2, mask=wm)
                plsc.store_scatter(kept, [_splat_i(4), K2v], ca, mask=wm)
                plsc.store_scatter(selv, [_splat_i(t)], idxv, mask=wm)
                plsc.store_scatter(sscv, [_splat_i(t)], Mv, mask=wm)
                kept_s = jnp.any(keepv)
                done = kept_s | jnp.logical_not(jnp.any(validv))
                return (jnp.where(done, jnp.int32(1), jnp.int32(0)),
                        K2 + jnp.where(kept_s, jnp.int32(1), jnp.int32(0)))

            _, K3 = lax.while_loop(wcond, wbody, (jnp.int32(0), K))
            return K3

        lax.fori_loop(0, MAXD, step, jnp.int32(0))
        pltpu.sync_copy(selv, sel_hbm.at[img, klass])
        pltpu.sync_copy(sscv, ssc_hbm.at[img, klass])

    run_task(wid)

    @pl.when(wid + 32 < B * C)
    def _second():
        run_task(wid + 32)


def _merge_body(selsc_hbm, sel_hbm, boxes_hbm, ob_hbm, os_hbm, ol_hbm,
                sscv, selv, bxv, heads, outb, outs, outl):
    cid = lax.axis_index("c")
    sid = lax.axis_index("s")
    wid = sid * NC_ + cid
    lanes = _lanes()
    lane0 = lanes == 0

    @pl.when(wid < B)
    def _():
        img = wid
        pltpu.sync_copy(selsc_hbm.at[img], sscv)
        pltpu.sync_copy(sel_hbm.at[img], selv)
        pltpu.sync_copy(boxes_hbm.at[img], bxv)
        heads[pl.ds(0, L)] = jnp.zeros((L,), jnp.int32)
        heads[pl.ds(L, L)] = jnp.zeros((L,), jnp.int32)

        def io(i, carry):
            outs[pl.ds(i * L, L)] = jnp.full((L,), -1.0, jnp.float32)
            outl[pl.ds(i * L, L)] = jnp.full((L,), -1, jnp.int32)
            return carry

        lax.fori_loop(0, KV, io, 0)

        def iob(i, carry):
            outb[pl.ds(i * L, L)] = jnp.full((L,), -1.0, jnp.float32)
            return carry

        lax.fori_loop(0, MD_P * 4 // L, iob, 0)

        def step(t, carry):
            hl = plsc.load_gather(heads, [lanes])
            g_lo = plsc.load_gather(sscv, [lanes, hl])
            rhi = jnp.minimum(lanes + L, jnp.int32(C - 1))
            hh = plsc.load_gather(heads, [lanes + L])
            g_hi_raw = plsc.load_gather(sscv, [rhi, hh])
            g_hi = jnp.where(lanes < C - L, g_hi_raw, jnp.float32(-2.0))
            comb = jnp.maximum(g_lo, g_hi)
            Mv = _tree_max(comb)
            in_lo = _as_splat_i(
                plsc.all_reduce_population_count(g_lo == Mv)) > 0
            lane_lo = _as_splat_i(plsc.all_reduce_ffs(g_lo == Mv))
            lane_hi = _as_splat_i(plsc.all_reduce_ffs(g_hi == Mv))
            clsv = jnp.where(in_lo, lane_lo, lane_hi + jnp.int32(L))
            hv = plsc.load_gather(heads, [clsv])
            plsc.store_scatter(heads, [clsv], hv + 1, mask=lane0)
            bi = plsc.load_gather(selv, [clsv, hv])
            vv = bi >= 0
            safe = jnp.maximum(bi, 0)
            coord = lax.rem(lanes, jnp.int32(4))
            bvals = plsc.load_gather(bxv, [safe, coord])
            obv = jnp.where(vv, bvals, jnp.float32(-1.0))
            plsc.store_scatter(outb, [_splat_i(t * 4) + coord], obv,
                               mask=lanes < 4)
            plsc.store_scatter(outs, [_splat_i(t)],
                               jnp.where(vv, Mv, jnp.float32(-1.0)),
                               mask=lane0)
            plsc.store_scatter(outl, [_splat_i(t)],
                               jnp.where(vv, clsv, jnp.int32(-1)), mask=lane0)
            return carry

        lax.fori_loop(0, MAXD, step, 0)
        pltpu.sync_copy(outb, ob_hbm.at[img])
        pltpu.sync_copy(outs, os_hbm.at[img])
        pltpu.sync_copy(outl, ol_hbm.at[img])


_mesh = plsc.VectorSubcoreMesh(core_axis_name="c", subcore_axis_name="s")

_nms_call = pl.kernel(
    _nms_body,
    out_type=(jax.ShapeDtypeStruct((B, C, MD_P), jnp.int32),
              jax.ShapeDtypeStruct((B, C, MD_P), jnp.float32)),
    mesh=_mesh,
    compiler_params=pltpu.CompilerParams(needs_layout_passes=False, use_tc_tiling_on_sc=False),
    scratch_types=[
        pltpu.VMEM((NP,), jnp.float32),
        pltpu.VMEM((NP,), jnp.float32),
        pltpu.VMEM((4, NP), jnp.float32),
        pltpu.VMEM((NP,), jnp.float32),
        pltpu.VMEM((CMAXP,), jnp.float32),
        pltpu.VMEM((5, MD_P + L), jnp.float32),
        pltpu.VMEM((MD_P,), jnp.int32),
        pltpu.VMEM((MD_P,), jnp.float32),
    ],
)

_merge_call = pl.kernel(
    _merge_body,
    out_type=(jax.ShapeDtypeStruct((B, MD_P * 4), jnp.float32),
              jax.ShapeDtypeStruct((B, MD_P), jnp.float32),
              jax.ShapeDtypeStruct((B, MD_P), jnp.int32)),
    mesh=_mesh,
    compiler_params=pltpu.CompilerParams(needs_layout_passes=False, use_tc_tiling_on_sc=False),
    scratch_types=[
        pltpu.VMEM((C, MD_P), jnp.float32),
        pltpu.VMEM((C, MD_P), jnp.int32),
        pltpu.VMEM((N, 4), jnp.float32),
        pltpu.VMEM((2 * L,), jnp.int32),
        pltpu.VMEM((MD_P * 4,), jnp.float32),
        pltpu.VMEM((MD_P,), jnp.float32),
        pltpu.VMEM((MD_P,), jnp.int32),
    ],
)


def kernel(boxes, classification):
    b = boxes.astype(jnp.float32)
    c = classification.astype(jnp.float32)
    scores_t = jnp.pad(jnp.transpose(c, (0, 2, 1)),
                       ((0, 0), (0, 0), (0, NP - N)))
    boxes_t = jnp.pad(jnp.transpose(b, (0, 2, 1)),
                      ((0, 0), (0, 0), (0, NP - N)))
    sel, ssc = _nms_call(scores_t, boxes_t)
    ob, osc, ol = _merge_call(ssc, sel, b)
    ob = ob.reshape(B, MD_P, 4)[:, :MAXD, :]
    return ob, osc[:, :MAXD], ol[:, :MAXD]


# speedup vs baseline: 65.1429x; 1.1537x over previous
import jax
import jax.numpy as jnp
import numpy as np
from jax import lax
from jax.experimental import pallas as pl
from jax.experimental.pallas import tpu as pltpu
from jax.experimental.pallas import tpu_sc as plsc

NEG = np.float32(-1e9)
SCORE_THR = np.float32(0.05)
HALF = np.float32(0.5)
VALID_CUT = np.float32(-5e8)
BIG = np.int32(1 << 30)

N = 5000
NP = 5120
L = 16
NCH = NP // L
CMAXP = 320
NCV = CMAXP // L
C = 20
B = 2
MAXD = 300
MD_P = 304
KV = MD_P // L
NC_ = 2
NS_ = 16


def _lanes():
    return lax.broadcasted_iota(jnp.int32, (L,), 0)


def _splat_i(x):
    return jnp.zeros((L,), jnp.int32) + x


def _splat_f(x):
    return jnp.zeros((L,), jnp.float32) + x


def _scalar(x):
    return x if x.ndim == 0 else jnp.max(x)


def _as_splat_i(x):
    return _splat_i(x) if x.ndim == 0 else x


def _rot(v, sh):
    lanes = _lanes()
    idx = jnp.bitwise_and(lanes + sh, L - 1)
    return v.at[idx].get(mode="promise_in_bounds")


def _tree_max(v):
    for sh in (8, 4, 2, 1):
        v = jnp.maximum(v, _rot(v, sh))
    return v


def _tree_min(v):
    for sh in (8, 4, 2, 1):
        v = jnp.minimum(v, _rot(v, sh))
    return v


def _nms_body(scores_hbm, boxes_hbm, ob_hbm, os_hbm, ol_hbm, *refs):
    bufs0 = refs[0:8]
    bufs1 = refs[8:16]
    sh_sel, sh_ssc, mg_sel, mg_ssc, heads, outb, outs, outl = refs[16:24]
    cid = lax.axis_index("c")
    sid = lax.axis_index("s")
    wid = sid * NC_ + cid
    lanes = _lanes()
    lane0 = lanes == 0

    def load_task(task, bufs):
        sraw, s, bx, area, cmax, kept, selv, sscv = bufs
        img = task // C
        klass = lax.rem(task, C)
        pltpu.sync_copy(scores_hbm.at[img, klass], sraw)
        pltpu.sync_copy(boxes_hbm.at[img], bx)

        def init_out(i, carry):
            @pl.when(i < KV)
            def _():
                selv[pl.ds(i * L, L)] = jnp.full((L,), -1, jnp.int32)
                sscv[pl.ds(i * L, L)] = jnp.full((L,), -1.0, jnp.float32)
            kept[0, pl.ds(i * L, L)] = jnp.full((L,), 3.0, jnp.float32)
            kept[1, pl.ds(i * L, L)] = jnp.full((L,), 3.0, jnp.float32)
            kept[2, pl.ds(i * L, L)] = jnp.full((L,), 0.0, jnp.float32)
            kept[3, pl.ds(i * L, L)] = jnp.full((L,), 0.0, jnp.float32)
            kept[4, pl.ds(i * L, L)] = jnp.full((L,), 0.0, jnp.float32)
            return carry

        lax.fori_loop(0, KV + 1, init_out, 0)

        def init_chunk(i, carry):
            for u in range(4):
                c = i * 4 + u
                v = sraw[pl.ds(c * L, L)]
                m = jnp.where(v > SCORE_THR, v, NEG)
                s[pl.ds(c * L, L)] = m
                y1 = bx[0, pl.ds(c * L, L)]
                x1 = bx[1, pl.ds(c * L, L)]
                y2 = bx[2, pl.ds(c * L, L)]
                x2 = bx[3, pl.ds(c * L, L)]
                a = jnp.maximum(y2 - y1, jnp.float32(0.0)) * \
                    jnp.maximum(x2 - x1, jnp.float32(0.0))
                area[pl.ds(c * L, L)] = a
                plsc.store_scatter(cmax, [_splat_i(c)], _splat_f(jnp.max(m)),
                                   mask=lane0)
            return carry

        lax.fori_loop(0, NCH // 4, init_chunk, 0)

    def pop(bufs, finv):
        sraw, s, bx, area, cmax, kept, selv, sscv = bufs
        m = cmax[pl.ds(0, L)]
        ci = jnp.zeros((L,), jnp.int32)
        for i in range(1, NCV):
            v = cmax[pl.ds(i * L, L)]
            upd = v > m
            m = jnp.where(upd, v, m)
            ci = jnp.where(upd, _splat_i(i), ci)
        Mv = _tree_max(m)
        gc = jnp.where(m == Mv, ci * L + lanes, BIG)
        cstarv = _tree_min(gc)
        validv = Mv > VALID_CUT

        v = plsc.load_gather(s, [cstarv * L + lanes])
        lanev = _as_splat_i(plsc.all_reduce_ffs(v == Mv))
        idxv = cstarv * L + lanev
        cy1 = plsc.load_gather(bx, [jnp.zeros((L,), jnp.int32), idxv])
        cx1 = plsc.load_gather(bx, [_splat_i(1), idxv])
        cy2 = plsc.load_gather(bx, [_splat_i(2), idxv])
        cx2 = plsc.load_gather(bx, [_splat_i(3), idxv])
        ca = plsc.load_gather(area, [idxv])
        rm = lane0 & validv & jnp.logical_not(finv)
        plsc.store_scatter(s, [idxv],
                           jnp.full((L,), NEG, jnp.float32), mask=rm)
        v2 = jnp.where(lanes == lanev, NEG, v)
        plsc.store_scatter(cmax, [cstarv], _tree_max(v2), mask=rm)
        return (cy1, cx1, cy2, cx2, ca, idxv, Mv, validv)

    def iou_block(cand, keptref, off, acc):
        cy1, cx1, cy2, cx2, ca = cand
        ky1 = keptref[0, pl.ds(off, L)]
        kx1 = keptref[1, pl.ds(off, L)]
        ky2 = keptref[2, pl.ds(off, L)]
        kx2 = keptref[3, pl.ds(off, L)]
        ka = keptref[4, pl.ds(off, L)]
        yy1 = jnp.maximum(cy1, ky1)
        xx1 = jnp.maximum(cx1, kx1)
        yy2 = jnp.minimum(cy2, ky2)
        xx2 = jnp.minimum(cx2, kx2)
        inter = jnp.maximum(yy2 - yy1, jnp.float32(0.0)) * \
            jnp.maximum(xx2 - xx1, jnp.float32(0.0))
        rhs = HALF * ((ca + ka) - inter)
        return jnp.logical_or(acc, inter > rhs)

    def commit(bufs, p, supv, t, K, fin):
        sraw, s, bx, area, cmax, kept, selv, sscv = bufs
        cy1, cx1, cy2, cx2, ca, idxv, Mv, validv = p
        keepv = validv & jnp.logical_not(supv)
        finv = _splat_i(fin) > 0
        wm = lane0 & keepv & jnp.logical_not(finv)
        Kv = _splat_i(K)
        plsc.store_scatter(kept, [jnp.zeros((L,), jnp.int32), Kv], cy1,
                           mask=wm)
        plsc.store_scatter(kept, [_splat_i(1), Kv], cx1, mask=wm)
        plsc.store_scatter(kept, [_splat_i(2), Kv], cy2, mask=wm)
        plsc.store_scatter(kept, [_splat_i(3), Kv], cx2, mask=wm)
        plsc.store_scatter(kept, [_splat_i(4), Kv], ca, mask=wm)
        plsc.store_scatter(selv, [_splat_i(t)], idxv, mask=wm)
        plsc.store_scatter(sscv, [_splat_i(t)], Mv, mask=wm)
        return jnp.any(keepv), jnp.any(validv)

    def store_task(task, bufs):
        sraw, s, bx, area, cmax, kept, selv, sscv = bufs
        klass = lax.rem(task, C)
        pltpu.sync_copy(selv, sh_sel.at[klass])
        pltpu.sync_copy(sscv, sh_ssc.at[klass])

    task0 = cid * C + sid
    task1 = cid * C + jnp.minimum(sid + NS_, C - 1)
    load_task(task0, bufs0)

    def single(_):
        def wcond(cw):
            t, K, alive = cw
            return (t < MAXD) & (alive > 0)

        def wbody(cw):
            t, K, alive = cw
            nofin = jnp.zeros((L,), jnp.bool_)
            p = pop(bufs0, nofin)
            cand = p[0:5]
            keptref = bufs0[5]

            def vb(j, acc):
                acc = iou_block(cand, keptref, j * 2 * L, acc)
                return iou_block(cand, keptref, j * 2 * L + L, acc)

            acc = lax.fori_loop(0, (K + 31) // 32, vb,
                                jnp.zeros((L,), jnp.bool_))
            supv = _as_splat_i(plsc.all_reduce_population_count(acc)) > 0
            kept_s, valid_s = commit(bufs0, p, supv, t, K, jnp.int32(0))
            inc = jnp.where(kept_s, jnp.int32(1), jnp.int32(0))
            return (t + inc, K + inc,
                    jnp.where(valid_s, alive, jnp.int32(0)))

        lax.while_loop(wcond, wbody, (jnp.int32(0), jnp.int32(0),
                                      jnp.int32(1)))
        return 0

    def dual(_):
        load_task(task1, bufs1)

        def wcond(cw):
            t0, K0, a0, t1, K1, a1 = cw
            return ((t0 < MAXD) & (a0 > 0)) | ((t1 < MAXD) & (a1 > 0))

        def wbody(cw):
            t0, K0, a0, t1, K1, a1 = cw
            fin0 = jnp.where((t0 < MAXD) & (a0 > 0), jnp.int32(0),
                             jnp.int32(1))
            fin1 = jnp.where((t1 < MAXD) & (a1 > 0), jnp.int32(0),
                             jnp.int32(1))
            fin0v = _splat_i(fin0) > 0
            fin1v = _splat_i(fin1) > 0
            p0 = pop(bufs0, fin0v)
            p1 = pop(bufs1, fin1v)
            cand0 = p0[0:5]
            cand1 = p1[0:5]
            kept0 = bufs0[5]
            kept1 = bufs1[5]

            def vb(j, accs):
                a, b = accs
                a = iou_block(cand0, kept0, j * 2 * L, a)
                b = iou_block(cand1, kept1, j * 2 * L, b)
                a = iou_block(cand0, kept0, j * 2 * L + L, a)
                b = iou_block(cand1, kept1, j * 2 * L + L, b)
                return (a, b)

            acc0, acc1 = lax.fori_loop(
                0, (jnp.maximum(K0, K1) + 31) // 32, vb,
                (jnp.zeros((L,), jnp.bool_), jnp.zeros((L,), jnp.bool_)))
            sup0 = _as_splat_i(plsc.all_reduce_population_count(acc0)) > 0
            sup1 = _as_splat_i(plsc.all_reduce_population_count(acc1)) > 0
            k0, v0 = commit(bufs0, p0, sup0, t0, K0, fin0)
            k1, v1 = commit(bufs1, p1, sup1, t1, K1, fin1)
            inc0 = jnp.where(k0 & (fin0 == 0), jnp.int32(1), jnp.int32(0))
            inc1 = jnp.where(k1 & (fin1 == 0), jnp.int32(1), jnp.int32(0))
            a0n = jnp.where((fin0 == 0) & jnp.logical_not(v0), jnp.int32(0),
                            a0)
            a1n = jnp.where((fin1 == 0) & jnp.logical_not(v1), jnp.int32(0),
                            a1)
            return (t0 + inc0, K0 + inc0, a0n, t1 + inc1, K1 + inc1, a1n)

        lax.while_loop(wcond, wbody,
                       (jnp.int32(0), jnp.int32(0), jnp.int32(1),
                        jnp.int32(0), jnp.int32(0), jnp.int32(1)))
        store_task(task1, bufs1)
        return 0

    lax.cond(sid + NS_ < C, dual, single, 0)
    store_task(task0, bufs0)
    plsc.subcore_barrier()

    @pl.when(sid == NS_ - 1)
    def _merge():
        img = cid
        bx = bufs0[2]
        pltpu.sync_copy(sh_ssc, mg_ssc)
        pltpu.sync_copy(sh_sel, mg_sel)
        heads[pl.ds(0, L)] = jnp.zeros((L,), jnp.int32)
        heads[pl.ds(L, L)] = jnp.zeros((L,), jnp.int32)

        def io(i, carry):
            outs[pl.ds(i * L, L)] = jnp.full((L,), -1.0, jnp.float32)
            outl[pl.ds(i * L, L)] = jnp.full((L,), -1, jnp.int32)
            return carry

        lax.fori_loop(0, KV, io, 0)

        def iob(i, carry):
            outb[pl.ds(i * L, L)] = jnp.full((L,), -1.0, jnp.float32)
            return carry

        lax.fori_loop(0, MD_P * 4 // L, iob, 0)

        def mstep(t, carry):
            hl = plsc.load_gather(heads, [lanes])
            g_lo = plsc.load_gather(mg_ssc, [lanes, hl])
            rhi = jnp.minimum(lanes + L, jnp.int32(C - 1))
            hh = plsc.load_gather(heads, [lanes + L])
            g_hi_raw = plsc.load_gather(mg_ssc, [rhi, hh])
            g_hi = jnp.where(lanes < C - L, g_hi_raw, jnp.float32(-2.0))
            comb = jnp.maximum(g_lo, g_hi)
            Mv = _tree_max(comb)
            in_lo = _as_splat_i(
                plsc.all_reduce_population_count(g_lo == Mv)) > 0
            lane_lo = _as_splat_i(plsc.all_reduce_ffs(g_lo == Mv))
            lane_hi = _as_splat_i(plsc.all_reduce_ffs(g_hi == Mv))
            clsv = jnp.where(in_lo, lane_lo, lane_hi + jnp.int32(L))
            hv = plsc.load_gather(heads, [clsv])
            plsc.store_scatter(heads, [clsv], hv + 1, mask=lane0)
            bi = plsc.load_gather(mg_sel, [clsv, hv])
            vv = bi >= 0
            safe = jnp.maximum(bi, 0)
            coord = lax.rem(lanes, jnp.int32(4))
            bvals = plsc.load_gather(bx, [coord, safe])
            obv = jnp.where(vv, bvals, jnp.float32(-1.0))
            plsc.store_scatter(outb, [_splat_i(t * 4) + coord], obv,
                               mask=lanes < 4)
            plsc.store_scatter(outs, [_splat_i(t)],
                               jnp.where(vv, Mv, jnp.float32(-1.0)),
                               mask=lane0)
            plsc.store_scatter(outl, [_splat_i(t)],
                               jnp.where(vv, clsv, jnp.int32(-1)), mask=lane0)
            return carry

        lax.fori_loop(0, MAXD, mstep, 0)
        pltpu.sync_copy(outb, ob_hbm.at[img])
        pltpu.sync_copy(outs, os_hbm.at[img])
        pltpu.sync_copy(outl, ol_hbm.at[img])


_mesh = plsc.VectorSubcoreMesh(core_axis_name="c", subcore_axis_name="s")

_nms_call = pl.kernel(
    _nms_body,
    out_type=(jax.ShapeDtypeStruct((B, MD_P * 4), jnp.float32),
              jax.ShapeDtypeStruct((B, MD_P), jnp.float32),
              jax.ShapeDtypeStruct((B, MD_P), jnp.int32)),
    mesh=_mesh,
    compiler_params=pltpu.CompilerParams(needs_layout_passes=False,
                                         use_tc_tiling_on_sc=False),
    scratch_types=[
        pltpu.VMEM((NP,), jnp.float32),
        pltpu.VMEM((NP,), jnp.float32),
        pltpu.VMEM((4, NP), jnp.float32),
        pltpu.VMEM((NP,), jnp.float32),
        pltpu.VMEM((CMAXP,), jnp.float32),
        pltpu.VMEM((5, MD_P + L), jnp.float32),
        pltpu.VMEM((MD_P,), jnp.int32),
        pltpu.VMEM((MD_P,), jnp.float32),
    ] * 2 + [
        pltpu.VMEM_SHARED((C, MD_P), jnp.int32),
        pltpu.VMEM_SHARED((C, MD_P), jnp.float32),
        pltpu.VMEM((C, MD_P), jnp.int32),
        pltpu.VMEM((C, MD_P), jnp.float32),
        pltpu.VMEM((2 * L,), jnp.int32),
        pltpu.VMEM((MD_P * 4,), jnp.float32),
        pltpu.VMEM((MD_P,), jnp.float32),
        pltpu.VMEM((MD_P,), jnp.int32),
    ],
)


def kernel(boxes, classification):
    b = boxes.astype(jnp.float32)
    c = classification.astype(jnp.float32)
    scores_t = jnp.pad(jnp.transpose(c, (0, 2, 1)),
                       ((0, 0), (0, 0), (0, NP - N)))
    boxes_t = jnp.pad(jnp.transpose(b, (0, 2, 1)),
                      ((0, 0), (0, 0), (0, NP - N)))
    ob, osc, ol = _nms_call(scores_t, boxes_t)
    ob = ob.reshape(B, MD_P, 4)[:, :MAXD, :]
    return ob, osc[:, :MAXD], ol[:, :MAXD]
